# baseline (speedup 1.0000x reference)
import functools

import jax
import jax.numpy as jnp
from jax import lax
from jax.experimental import pallas as pl
from jax.experimental.pallas import tpu as pltpu
from jax.experimental.pallas import tpu_sc as plsc


def _sc_gather(row_flat, col_flat, aux, ns_am, NW, PPW, AUXW, R, C, D,
               two_boffs):
    P = NW * PPW
    CH = 64
    NCH = PPW // CH
    GPC = CH // 16
    NSEG = 4 if two_boffs else 3
    DG = D // 16

    mesh = plsc.VectorSubcoreMesh(core_axis_name="c", subcore_axis_name="s")

    @functools.partial(
        pl.kernel,
        mesh=mesh,
        out_type=[
            jax.ShapeDtypeStruct((P, D), jnp.float32),
            jax.ShapeDtypeStruct((P, D), jnp.float32),
        ],
        scratch_types=[
            pltpu.VMEM((AUXW,), jnp.int32),
            pltpu.VMEM((PPW,), jnp.int32),
            pltpu.VMEM((NCH, CH), jnp.int32),
            pltpu.VMEM((NCH, CH), jnp.int32),
            pltpu.VMEM((PPW,), jnp.float32),
            pltpu.VMEM((PPW,), jnp.float32),
            pltpu.VMEM((PPW, D), jnp.float32),
            pltpu.VMEM((PPW, D), jnp.float32),
            pltpu.SemaphoreType.DMA,
            pltpu.SemaphoreType.DMA,
            pltpu.SemaphoreType.DMA,
            pltpu.SemaphoreType.DMA,
        ],
    )
    def sc_kernel(row_hbm, col_hbm, aux_hbm, ns_hbm, lot_out, col_out,
                  aux_v, ns_v, lotg_v, colg_v, mlot_v, mcol_v,
                  lotrows_v, colrows_v, sem0, sem1, sem2, sem3):
        wid = lax.axis_index("s") * 2 + lax.axis_index("c")
        base = wid * PPW

        aux_cp = pltpu.async_copy(aux_hbm.at[pl.ds(wid * AUXW, AUXW)], aux_v,
                                  sem2)
        ns_cp = pltpu.async_copy(ns_hbm.at[pl.ds(base, PPW)], ns_v, sem3)
        aux_cp.wait()
        nlt = aux_v[pl.ds(NSEG * PPW, 16)]
        nst = aux_v[pl.ds(NSEG * PPW + 16, 16)]

        lot_cps, col_cps = [], []
        for j in range(NCH):
            for gc in range(GPC):
                s = j * CH + gc * 16
                r = gc * 16
                idx = aux_v[pl.ds(s, 16)]
                vlot = idx <= nlt
                safe_lot = jnp.minimum(jnp.where(vlot, idx, 0), R - 1)
                lotg_v[j, pl.ds(r, 16)] = (aux_v[pl.ds(2 * PPW + s, 16)]
                                           + safe_lot)
                mlot_v[pl.ds(s, 16)] = jnp.where(vlot, 1.0, 0.0)
            lot_cps.append(pltpu.async_copy(
                row_hbm.at[lotg_v.at[j]],
                lotrows_v.at[pl.ds(j * CH, CH)], sem0))

        ns_cp.wait()
        for j in range(NCH):
            for gc in range(GPC):
                s = j * CH + gc * 16
                r = gc * 16
                stp = aux_v[pl.ds(PPW + s, 16)]
                ns = ns_v[pl.ds(s, 16)]
                boffc = (aux_v[pl.ds(3 * PPW + s, 16)] if two_boffs
                         else aux_v[pl.ds(2 * PPW + s, 16)])
                vcol = jnp.logical_and(stp + 1 <= nst,
                                       jnp.logical_and(ns >= 1, ns <= C))
                safe_col = jnp.where(vcol, ns - 1, 0)
                colg_v[j, pl.ds(r, 16)] = boffc + safe_col
                mcol_v[pl.ds(s, 16)] = jnp.where(vcol, 1.0, 0.0)
            col_cps.append(pltpu.async_copy(
                col_hbm.at[colg_v.at[j]],
                colrows_v.at[pl.ds(j * CH, CH)], sem1))

        zeros16 = jnp.zeros((16,), jnp.float32)

        def make_zero_scan(mask_v, rows_v):
            def zero_scan(g, carry):
                s16 = g * 16
                m16 = mask_v[pl.ds(s16, 16)]
                for l in range(16):
                    @pl.when(m16[l] == 0.0)
                    def _():
                        for gg in range(DG):
                            rows_v[s16 + l, pl.ds(gg * 16, 16)] = zeros16
                return carry
            return zero_scan

        wbs = []
        for j in range(NCH):
            lot_cps[j].wait()
            lax.fori_loop(j * GPC, (j + 1) * GPC,
                          make_zero_scan(mlot_v, lotrows_v), 0)
            wbs.append(pltpu.async_copy(
                lotrows_v.at[pl.ds(j * CH, CH)],
                lot_out.at[pl.ds(base + j * CH, CH)], sem2))
        for j in range(NCH):
            col_cps[j].wait()
            lax.fori_loop(j * GPC, (j + 1) * GPC,
                          make_zero_scan(mcol_v, colrows_v), 0)
            wbs.append(pltpu.async_copy(
                colrows_v.at[pl.ds(j * CH, CH)],
                col_out.at[pl.ds(base + j * CH, CH)], sem3))
        for wb in wbs:
            wb.wait()

    return sc_kernel(row_flat, col_flat, aux, ns_am)


def _tc_combine(lot_rows, col_rows, W, B, D):
    BB = 512
    grid = (B // BB,)
    dn = (((1,), (1,)), ((), ()))

    def body(lot_ref, col_ref, w_ref, out_ref):
        e0 = lot_ref[0] + col_ref[0]
        e1 = lot_ref[1] + col_ref[1]
        out_ref[...] = (
            lax.dot_general(e0, w_ref[:, :D], dn,
                            preferred_element_type=jnp.float32)
            + lax.dot_general(e1, w_ref[:, D:], dn,
                              preferred_element_type=jnp.float32))

    return pl.pallas_call(
        body,
        grid=grid,
        in_specs=[
            pl.BlockSpec((2, BB, D), lambda i: (0, i, 0)),
            pl.BlockSpec((2, BB, D), lambda i: (0, i, 0)),
            pl.BlockSpec((D, 2 * D), lambda i: (0, 0)),
        ],
        out_specs=pl.BlockSpec((BB, D), lambda i: (i, 0)),
        out_shape=jax.ShapeDtypeStruct((B, D), jnp.float32),
    )(lot_rows, col_rows, W)


def kernel(encoded_row, encoded_col, W, robot_lot_idx, robot_lot_step, flow,
           num_lot_type, num_step):
    B, R, D = encoded_row.shape
    C = encoded_col.shape[1]

    row_flat = encoded_row.reshape(B * R, D)
    col_flat = encoded_col.reshape(B * C, D)
    idx_am = robot_lot_idx.T.reshape(-1).astype(jnp.int32)
    step_am = robot_lot_step.T.reshape(-1).astype(jnp.int32)

    b_am = jnp.tile(jnp.arange(B, dtype=jnp.int32), 2)
    dns_am = jnp.where(step_am + 1 > num_step, 0, step_am + 1)
    lot_f_am = jnp.where(idx_am <= num_lot_type, idx_am, 0)
    ns_am = flow[b_am, lot_f_am, dns_am].astype(jnp.int32)

    info = plsc.get_sparse_core_info()
    NW = info.num_cores * info.num_subcores
    P = 2 * B
    PPW = P // NW
    segs = [idx_am, step_am, b_am * R]
    if R != C:
        segs.append(b_am * C)
    AUXW = len(segs) * PPW + 32
    aux = jnp.concatenate(
        [jnp.stack([s.reshape(NW, PPW) for s in segs], axis=1).reshape(
            NW, len(segs) * PPW),
         jnp.full((NW, 16), num_lot_type, jnp.int32),
         jnp.full((NW, 16), num_step, jnp.int32)], axis=1).reshape(-1)

    lot_rows, col_rows = _sc_gather(
        row_flat, col_flat, aux, ns_am, NW, PPW, AUXW, R, C, D, R != C)

    return _tc_combine(lot_rows.reshape(2, B, D), col_rows.reshape(2, B, D),
                       W, B, D)


# speedup vs baseline: 1.0374x; 1.0374x over previous
import functools

import jax
import jax.numpy as jnp
from jax import lax
from jax.experimental import pallas as pl
from jax.experimental.pallas import tpu as pltpu
from jax.experimental.pallas import tpu_sc as plsc


def _sc_gather(row_flat, col_flat, aux, ns_am, NW, PPW, AUXW, R, C, D,
               two_boffs):
    P = NW * PPW
    CH = 128
    NCH = PPW // CH
    GPC = CH // 16
    NSEG = 4 if two_boffs else 3
    DG = D // 16

    mesh = plsc.VectorSubcoreMesh(core_axis_name="c", subcore_axis_name="s")

    @functools.partial(
        pl.kernel,
        mesh=mesh,
        out_type=[
            jax.ShapeDtypeStruct((P, D), jnp.float32),
            jax.ShapeDtypeStruct((P, D), jnp.float32),
        ],
        scratch_types=[
            pltpu.VMEM((AUXW,), jnp.int32),
            pltpu.VMEM((PPW,), jnp.int32),
            pltpu.VMEM((NCH, CH), jnp.int32),
            pltpu.VMEM((NCH, CH), jnp.int32),
            pltpu.VMEM((PPW,), jnp.float32),
            pltpu.VMEM((PPW,), jnp.float32),
            pltpu.VMEM((PPW, D), jnp.float32),
            pltpu.VMEM((PPW, D), jnp.float32),
            pltpu.SemaphoreType.DMA,
            pltpu.SemaphoreType.DMA,
            pltpu.SemaphoreType.DMA,
            pltpu.SemaphoreType.DMA,
        ],
    )
    def sc_kernel(row_hbm, col_hbm, aux_hbm, ns_hbm, lot_out, col_out,
                  aux_v, ns_v, lotg_v, colg_v, mlot_v, mcol_v,
                  lotrows_v, colrows_v, sem0, sem1, sem2, sem3):
        wid = lax.axis_index("s") * 2 + lax.axis_index("c")
        base = wid * PPW

        aux_cp = pltpu.async_copy(aux_hbm.at[pl.ds(wid * AUXW, AUXW)], aux_v,
                                  sem2)
        ns_cp = pltpu.async_copy(ns_hbm.at[pl.ds(base, PPW)], ns_v, sem3)
        aux_cp.wait()
        nlt = aux_v[pl.ds(NSEG * PPW, 16)]
        nst = aux_v[pl.ds(NSEG * PPW + 16, 16)]

        lot_cps, col_cps = [], []
        for j in range(NCH):
            for gc in range(GPC):
                s = j * CH + gc * 16
                r = gc * 16
                idx = aux_v[pl.ds(s, 16)]
                vlot = idx <= nlt
                safe_lot = jnp.minimum(jnp.where(vlot, idx, 0), R - 1)
                lotg_v[j, pl.ds(r, 16)] = (aux_v[pl.ds(2 * PPW + s, 16)]
                                           + safe_lot)
                mlot_v[pl.ds(s, 16)] = jnp.where(vlot, 1.0, 0.0)
            lot_cps.append(pltpu.async_copy(
                row_hbm.at[lotg_v.at[j]],
                lotrows_v.at[pl.ds(j * CH, CH)], sem0))

        ns_cp.wait()
        for j in range(NCH):
            for gc in range(GPC):
                s = j * CH + gc * 16
                r = gc * 16
                stp = aux_v[pl.ds(PPW + s, 16)]
                ns = ns_v[pl.ds(s, 16)]
                boffc = (aux_v[pl.ds(3 * PPW + s, 16)] if two_boffs
                         else aux_v[pl.ds(2 * PPW + s, 16)])
                vcol = jnp.logical_and(stp + 1 <= nst,
                                       jnp.logical_and(ns >= 1, ns <= C))
                safe_col = jnp.where(vcol, ns - 1, 0)
                colg_v[j, pl.ds(r, 16)] = boffc + safe_col
                mcol_v[pl.ds(s, 16)] = jnp.where(vcol, 1.0, 0.0)
            col_cps.append(pltpu.async_copy(
                col_hbm.at[colg_v.at[j]],
                colrows_v.at[pl.ds(j * CH, CH)], sem1))

        zeros16 = jnp.zeros((16,), jnp.float32)

        def make_zero_scan(mask_v, rows_v):
            def zero_scan(g, carry):
                s16 = g * 16
                m16 = mask_v[pl.ds(s16, 16)]
                for l in range(16):
                    @pl.when(m16[l] == 0.0)
                    def _():
                        for gg in range(DG):
                            rows_v[s16 + l, pl.ds(gg * 16, 16)] = zeros16
                return carry
            return zero_scan

        wbs = []
        for j in range(NCH):
            lot_cps[j].wait()
            lax.fori_loop(j * GPC, (j + 1) * GPC,
                          make_zero_scan(mlot_v, lotrows_v), 0)
            wbs.append(pltpu.async_copy(
                lotrows_v.at[pl.ds(j * CH, CH)],
                lot_out.at[pl.ds(base + j * CH, CH)], sem2))
        for j in range(NCH):
            col_cps[j].wait()
            lax.fori_loop(j * GPC, (j + 1) * GPC,
                          make_zero_scan(mcol_v, colrows_v), 0)
            wbs.append(pltpu.async_copy(
                colrows_v.at[pl.ds(j * CH, CH)],
                col_out.at[pl.ds(base + j * CH, CH)], sem3))
        for wb in wbs:
            wb.wait()

    return sc_kernel(row_flat, col_flat, aux, ns_am)


def _tc_combine(lot_rows, col_rows, W, B, D):
    BB = 512
    grid = (B // BB,)
    dn = (((1,), (1,)), ((), ()))

    def body(lot_ref, col_ref, w_ref, out_ref):
        e0 = lot_ref[0] + col_ref[0]
        e1 = lot_ref[1] + col_ref[1]
        out_ref[...] = (
            lax.dot_general(e0, w_ref[:, :D], dn,
                            preferred_element_type=jnp.float32)
            + lax.dot_general(e1, w_ref[:, D:], dn,
                              preferred_element_type=jnp.float32))

    return pl.pallas_call(
        body,
        grid=grid,
        in_specs=[
            pl.BlockSpec((2, BB, D), lambda i: (0, i, 0)),
            pl.BlockSpec((2, BB, D), lambda i: (0, i, 0)),
            pl.BlockSpec((D, 2 * D), lambda i: (0, 0)),
        ],
        out_specs=pl.BlockSpec((BB, D), lambda i: (i, 0)),
        out_shape=jax.ShapeDtypeStruct((B, D), jnp.float32),
    )(lot_rows, col_rows, W)


def kernel(encoded_row, encoded_col, W, robot_lot_idx, robot_lot_step, flow,
           num_lot_type, num_step):
    B, R, D = encoded_row.shape
    C = encoded_col.shape[1]

    row_flat = encoded_row.reshape(B * R, D)
    col_flat = encoded_col.reshape(B * C, D)
    idx_am = robot_lot_idx.T.reshape(-1).astype(jnp.int32)
    step_am = robot_lot_step.T.reshape(-1).astype(jnp.int32)

    b_am = jnp.tile(jnp.arange(B, dtype=jnp.int32), 2)
    dns_am = jnp.where(step_am + 1 > num_step, 0, step_am + 1)
    lot_f_am = jnp.where(idx_am <= num_lot_type, idx_am, 0)
    ns_am = flow[b_am, lot_f_am, dns_am].astype(jnp.int32)

    info = plsc.get_sparse_core_info()
    NW = info.num_cores * info.num_subcores
    P = 2 * B
    PPW = P // NW
    segs = [idx_am, step_am, b_am * R]
    if R != C:
        segs.append(b_am * C)
    AUXW = len(segs) * PPW + 32
    aux = jnp.concatenate(
        [jnp.stack([s.reshape(NW, PPW) for s in segs], axis=1).reshape(
            NW, len(segs) * PPW),
         jnp.full((NW, 16), num_lot_type, jnp.int32),
         jnp.full((NW, 16), num_step, jnp.int32)], axis=1).reshape(-1)

    lot_rows, col_rows = _sc_gather(
        row_flat, col_flat, aux, ns_am, NW, PPW, AUXW, R, C, D, R != C)

    return _tc_combine(lot_rows.reshape(2, B, D), col_rows.reshape(2, B, D),
                       W, B, D)


# speedup vs baseline: 1.0909x; 1.0515x over previous
import functools

import jax
import jax.numpy as jnp
from jax import lax
from jax.experimental import pallas as pl
from jax.experimental.pallas import tpu as pltpu
from jax.experimental.pallas import tpu_sc as plsc


def _sc_gather(row_flat, col_flat, aux, ns_am, NW, PPW, AUXW, R, C, D,
               two_boffs):
    P = NW * PPW
    CH = 128
    NCH = PPW // CH
    GPC = CH // 16
    NSEG = 4 if two_boffs else 3
    DG = D // 16

    mesh = plsc.VectorSubcoreMesh(core_axis_name="c", subcore_axis_name="s")

    @functools.partial(
        pl.kernel,
        mesh=mesh,
        out_type=[
            jax.ShapeDtypeStruct((P, D), jnp.float32),
            jax.ShapeDtypeStruct((P, D), jnp.float32),
        ],
        scratch_types=[
            pltpu.VMEM((AUXW,), jnp.int32),
            pltpu.VMEM((PPW,), jnp.int32),
            pltpu.VMEM((NCH, CH), jnp.int32),
            pltpu.VMEM((NCH, CH), jnp.int32),
            pltpu.VMEM((PPW,), jnp.float32),
            pltpu.VMEM((PPW,), jnp.float32),
            pltpu.VMEM((PPW, D), jnp.float32),
            pltpu.VMEM((PPW, D), jnp.float32),
            pltpu.SemaphoreType.DMA,
            pltpu.SemaphoreType.DMA,
            pltpu.SemaphoreType.DMA,
            pltpu.SemaphoreType.DMA,
        ],
    )
    def sc_kernel(row_hbm, col_hbm, aux_hbm, ns_hbm, lot_out, col_out,
                  aux_v, ns_v, lotg_v, colg_v, mlot_v, mcol_v,
                  lotrows_v, colrows_v, sem0, sem1, sem2, sem3):
        wid = lax.axis_index("s") * 2 + lax.axis_index("c")
        base = wid * PPW

        aux_cp = pltpu.async_copy(aux_hbm.at[pl.ds(wid * AUXW, AUXW)], aux_v,
                                  sem2)
        ns_cp = pltpu.async_copy(ns_hbm.at[pl.ds(base, PPW)], ns_v, sem3)
        aux_cp.wait()
        nlt = aux_v[pl.ds(NSEG * PPW, 16)]
        nst = aux_v[pl.ds(NSEG * PPW + 16, 16)]

        lot_cps, col_cps = [], []
        for j in range(NCH):
            for gc in range(GPC):
                s = j * CH + gc * 16
                r = gc * 16
                idx = aux_v[pl.ds(s, 16)]
                vlot = idx <= nlt
                safe_lot = jnp.minimum(jnp.where(vlot, idx, 0), R - 1)
                lotg_v[j, pl.ds(r, 16)] = (aux_v[pl.ds(2 * PPW + s, 16)]
                                           + safe_lot)
                mlot_v[pl.ds(s, 16)] = jnp.where(vlot, 1.0, 0.0)
            lot_cps.append(pltpu.async_copy(
                row_hbm.at[lotg_v.at[j]],
                lotrows_v.at[pl.ds(j * CH, CH)], sem0))

        ns_cp.wait()
        for j in range(NCH):
            for gc in range(GPC):
                s = j * CH + gc * 16
                r = gc * 16
                stp = aux_v[pl.ds(PPW + s, 16)]
                ns = ns_v[pl.ds(s, 16)]
                boffc = (aux_v[pl.ds(3 * PPW + s, 16)] if two_boffs
                         else aux_v[pl.ds(2 * PPW + s, 16)])
                vcol = jnp.logical_and(stp + 1 <= nst,
                                       jnp.logical_and(ns >= 1, ns <= C))
                safe_col = jnp.where(vcol, ns - 1, 0)
                colg_v[j, pl.ds(r, 16)] = boffc + safe_col
                mcol_v[pl.ds(s, 16)] = jnp.where(vcol, 1.0, 0.0)
            col_cps.append(pltpu.async_copy(
                col_hbm.at[colg_v.at[j]],
                colrows_v.at[pl.ds(j * CH, CH)], sem1))

        zeros16 = jnp.zeros((16,), jnp.float32)

        def make_zero_scan(mask_v, rows_v):
            def zero_scan(g, carry):
                s16 = g * 16
                m16 = mask_v[pl.ds(s16, 16)]
                for l in range(16):
                    @pl.when(m16[l] == 0.0)
                    def _():
                        for gg in range(DG):
                            rows_v[s16 + l, pl.ds(gg * 16, 16)] = zeros16
                return carry
            return zero_scan

        wbs = []
        for j in range(NCH):
            lot_cps[j].wait()
            lax.fori_loop(j * GPC, (j + 1) * GPC,
                          make_zero_scan(mlot_v, lotrows_v), 0)
            wbs.append(pltpu.async_copy(
                lotrows_v.at[pl.ds(j * CH, CH)],
                lot_out.at[pl.ds(base + j * CH, CH)], sem2))
        for j in range(NCH):
            col_cps[j].wait()
            lax.fori_loop(j * GPC, (j + 1) * GPC,
                          make_zero_scan(mcol_v, colrows_v), 0)
            wbs.append(pltpu.async_copy(
                colrows_v.at[pl.ds(j * CH, CH)],
                col_out.at[pl.ds(base + j * CH, CH)], sem3))
        for wb in wbs:
            wb.wait()

    return sc_kernel(row_flat, col_flat, aux, ns_am)


def _tc_combine(lot_rows, col_rows, W, B, D):
    BB = 1024
    grid = (B // BB,)
    dn = (((1,), (1,)), ((), ()))

    def body(lot_ref, col_ref, w_ref, out_ref):
        e0 = lot_ref[0] + col_ref[0]
        e1 = lot_ref[1] + col_ref[1]
        out_ref[...] = (
            lax.dot_general(e0, w_ref[:, :D], dn,
                            preferred_element_type=jnp.float32)
            + lax.dot_general(e1, w_ref[:, D:], dn,
                              preferred_element_type=jnp.float32))

    return pl.pallas_call(
        body,
        grid=grid,
        in_specs=[
            pl.BlockSpec((2, BB, D), lambda i: (0, i, 0)),
            pl.BlockSpec((2, BB, D), lambda i: (0, i, 0)),
            pl.BlockSpec((D, 2 * D), lambda i: (0, 0)),
        ],
        out_specs=pl.BlockSpec((BB, D), lambda i: (i, 0)),
        out_shape=jax.ShapeDtypeStruct((B, D), jnp.float32),
    )(lot_rows, col_rows, W)


def kernel(encoded_row, encoded_col, W, robot_lot_idx, robot_lot_step, flow,
           num_lot_type, num_step):
    B, R, D = encoded_row.shape
    C = encoded_col.shape[1]

    row_flat = encoded_row.reshape(B * R, D)
    col_flat = encoded_col.reshape(B * C, D)
    idx_am = robot_lot_idx.T.reshape(-1).astype(jnp.int32)
    step_am = robot_lot_step.T.reshape(-1).astype(jnp.int32)

    b_am = jnp.tile(jnp.arange(B, dtype=jnp.int32), 2)
    dns_am = jnp.where(step_am + 1 > num_step, 0, step_am + 1)
    lot_f_am = jnp.where(idx_am <= num_lot_type, idx_am, 0)
    ns_am = flow[b_am, lot_f_am, dns_am].astype(jnp.int32)

    info = plsc.get_sparse_core_info()
    NW = info.num_cores * info.num_subcores
    P = 2 * B
    PPW = P // NW
    segs = [idx_am, step_am, b_am * R]
    if R != C:
        segs.append(b_am * C)
    AUXW = len(segs) * PPW + 32
    aux = jnp.concatenate(
        [jnp.stack([s.reshape(NW, PPW) for s in segs], axis=1).reshape(
            NW, len(segs) * PPW),
         jnp.full((NW, 16), num_lot_type, jnp.int32),
         jnp.full((NW, 16), num_step, jnp.int32)], axis=1).reshape(-1)

    lot_rows, col_rows = _sc_gather(
        row_flat, col_flat, aux, ns_am, NW, PPW, AUXW, R, C, D, R != C)

    return _tc_combine(lot_rows.reshape(2, B, D), col_rows.reshape(2, B, D),
                       W, B, D)


# speedup vs baseline: 1.1236x; 1.0300x over previous
import functools

import jax
import jax.numpy as jnp
from jax import lax
from jax.experimental import pallas as pl
from jax.experimental.pallas import tpu as pltpu
from jax.experimental.pallas import tpu_sc as plsc


def _sc_gather(row_flat, col_flat, aux, ns_am, NW, PPW, AUXW, R, C, D,
               two_boffs):
    P = NW * PPW
    CH = 128
    NCH = PPW // CH
    GPC = CH // 16
    NSEG = 4 if two_boffs else 3
    DG = D // 16

    mesh = plsc.VectorSubcoreMesh(core_axis_name="c", subcore_axis_name="s")

    @functools.partial(
        pl.kernel,
        mesh=mesh,
        out_type=[
            jax.ShapeDtypeStruct((P, D), jnp.float32),
            jax.ShapeDtypeStruct((P, D), jnp.float32),
        ],
        scratch_types=[
            pltpu.VMEM((AUXW,), jnp.int32),
            pltpu.VMEM((PPW,), jnp.int32),
            pltpu.VMEM((NCH, CH), jnp.int32),
            pltpu.VMEM((NCH, CH), jnp.int32),
            pltpu.VMEM((PPW,), jnp.float32),
            pltpu.VMEM((PPW,), jnp.float32),
            pltpu.VMEM((PPW, D), jnp.float32),
            pltpu.VMEM((PPW, D), jnp.float32),
            pltpu.SemaphoreType.DMA,
            pltpu.SemaphoreType.DMA,
            pltpu.SemaphoreType.DMA,
            pltpu.SemaphoreType.DMA,
        ],
    )
    def sc_kernel(row_hbm, col_hbm, aux_hbm, ns_hbm, lot_out, col_out,
                  aux_v, ns_v, lotg_v, colg_v, mlot_v, mcol_v,
                  lotrows_v, colrows_v, sem0, sem1, sem2, sem3):
        wid = lax.axis_index("s") * 2 + lax.axis_index("c")
        base = wid * PPW

        aux_cp = pltpu.async_copy(aux_hbm.at[pl.ds(wid * AUXW, AUXW)], aux_v,
                                  sem2)
        ns_cp = pltpu.async_copy(ns_hbm.at[pl.ds(base, PPW)], ns_v, sem3)
        aux_cp.wait()
        nlt = aux_v[pl.ds(NSEG * PPW, 16)]
        nst = aux_v[pl.ds(NSEG * PPW + 16, 16)]

        lot_cps, col_cps = [], []
        for j in range(NCH):
            for gc in range(GPC):
                s = j * CH + gc * 16
                r = gc * 16
                idx = aux_v[pl.ds(s, 16)]
                vlot = idx <= nlt
                safe_lot = jnp.minimum(jnp.where(vlot, idx, 0), R - 1)
                lotg_v[j, pl.ds(r, 16)] = (aux_v[pl.ds(2 * PPW + s, 16)]
                                           + safe_lot)
                mlot_v[pl.ds(s, 16)] = jnp.where(vlot, 1.0, 0.0)
            lot_cps.append(pltpu.async_copy(
                row_hbm.at[lotg_v.at[j]],
                lotrows_v.at[pl.ds(j * CH, CH)], sem0))

        ns_cp.wait()
        for j in range(NCH):
            for gc in range(GPC):
                s = j * CH + gc * 16
                r = gc * 16
                stp = aux_v[pl.ds(PPW + s, 16)]
                ns = ns_v[pl.ds(s, 16)]
                boffc = (aux_v[pl.ds(3 * PPW + s, 16)] if two_boffs
                         else aux_v[pl.ds(2 * PPW + s, 16)])
                vcol = jnp.logical_and(stp + 1 <= nst,
                                       jnp.logical_and(ns >= 1, ns <= C))
                safe_col = jnp.where(vcol, ns - 1, 0)
                colg_v[j, pl.ds(r, 16)] = boffc + safe_col
                mcol_v[pl.ds(s, 16)] = jnp.where(vcol, 1.0, 0.0)
            col_cps.append(pltpu.async_copy(
                col_hbm.at[colg_v.at[j]],
                colrows_v.at[pl.ds(j * CH, CH)], sem1))

        zeros16 = jnp.zeros((16,), jnp.float32)

        def make_zero_scan(mask_v, rows_v):
            def zero_scan(g, carry):
                s16 = g * 16
                m16 = mask_v[pl.ds(s16, 16)]
                for l in range(16):
                    @pl.when(m16[l] == 0.0)
                    def _():
                        for gg in range(DG):
                            rows_v[s16 + l, pl.ds(gg * 16, 16)] = zeros16
                return carry
            return zero_scan

        wbs = []
        for j in range(NCH):
            lot_cps[j].wait()
            lax.fori_loop(j * GPC, (j + 1) * GPC,
                          make_zero_scan(mlot_v, lotrows_v), 0)
            wbs.append(pltpu.async_copy(
                lotrows_v.at[pl.ds(j * CH, CH)],
                lot_out.at[pl.ds(base + j * CH, CH)], sem2))
        for j in range(NCH):
            col_cps[j].wait()
            lax.fori_loop(j * GPC, (j + 1) * GPC,
                          make_zero_scan(mcol_v, colrows_v), 0)
            wbs.append(pltpu.async_copy(
                colrows_v.at[pl.ds(j * CH, CH)],
                col_out.at[pl.ds(base + j * CH, CH)], sem3))
        for wb in wbs:
            wb.wait()

    return sc_kernel(row_flat, col_flat, aux, ns_am)


def _tc_combine(lot_rows, col_rows, W, B, D):
    BB = min(B, 2048)
    grid = (B // BB,)
    dn = (((1,), (1,)), ((), ()))

    def body(lot_ref, col_ref, w_ref, out_ref):
        e0 = lot_ref[0] + col_ref[0]
        e1 = lot_ref[1] + col_ref[1]
        out_ref[...] = (
            lax.dot_general(e0, w_ref[:, :D], dn,
                            preferred_element_type=jnp.float32)
            + lax.dot_general(e1, w_ref[:, D:], dn,
                              preferred_element_type=jnp.float32))

    return pl.pallas_call(
        body,
        grid=grid,
        in_specs=[
            pl.BlockSpec((2, BB, D), lambda i: (0, i, 0)),
            pl.BlockSpec((2, BB, D), lambda i: (0, i, 0)),
            pl.BlockSpec((D, 2 * D), lambda i: (0, 0)),
        ],
        out_specs=pl.BlockSpec((BB, D), lambda i: (i, 0)),
        out_shape=jax.ShapeDtypeStruct((B, D), jnp.float32),
    )(lot_rows, col_rows, W)


def kernel(encoded_row, encoded_col, W, robot_lot_idx, robot_lot_step, flow,
           num_lot_type, num_step):
    B, R, D = encoded_row.shape
    C = encoded_col.shape[1]

    row_flat = encoded_row.reshape(B * R, D)
    col_flat = encoded_col.reshape(B * C, D)
    idx_am = robot_lot_idx.T.reshape(-1).astype(jnp.int32)
    step_am = robot_lot_step.T.reshape(-1).astype(jnp.int32)

    b_am = jnp.tile(jnp.arange(B, dtype=jnp.int32), 2)
    dns_am = jnp.where(step_am + 1 > num_step, 0, step_am + 1)
    lot_f_am = jnp.where(idx_am <= num_lot_type, idx_am, 0)
    ns_am = flow[b_am, lot_f_am, dns_am].astype(jnp.int32)

    info = plsc.get_sparse_core_info()
    NW = info.num_cores * info.num_subcores
    P = 2 * B
    PPW = P // NW
    segs = [idx_am, step_am, b_am * R]
    if R != C:
        segs.append(b_am * C)
    AUXW = len(segs) * PPW + 32
    aux = jnp.concatenate(
        [jnp.stack([s.reshape(NW, PPW) for s in segs], axis=1).reshape(
            NW, len(segs) * PPW),
         jnp.full((NW, 16), num_lot_type, jnp.int32),
         jnp.full((NW, 16), num_step, jnp.int32)], axis=1).reshape(-1)

    lot_rows, col_rows = _sc_gather(
        row_flat, col_flat, aux, ns_am, NW, PPW, AUXW, R, C, D, R != C)

    return _tc_combine(lot_rows.reshape(2, B, D), col_rows.reshape(2, B, D),
                       W, B, D)
